# baseline (speedup 1.0000x reference)
import jax
import jax.numpy as jnp
from jax.experimental import pallas as pl
from jax.experimental.pallas import tpu as pltpu

_DS = 16
_MAT = 2048
_ND = _MAT - 2 * _DS
_R = 128
_BB = 32

_N_I = _MAT // _R
_MAXC = _N_I - 1
_MAX16 = _MAT // 16 - 1


def _vert16(x, n):
    w = x[:, 0:n - 1, :] + x[:, 1:n, :]
    w = w[:, 0:n - 3, :] + w[:, 2:n - 1, :]
    w = w[:, 0:n - 7, :] + w[:, 4:n - 3, :]
    w = w[:, 0:n - 15, :] + w[:, 8:n - 7, :]
    return w


def _chunk(xa, xc, xb, xd, sa, sb):
    dn = (((2,), (0,)), ((), ()))
    f32 = jnp.float32
    ha1 = jax.lax.dot_general(xa.astype(jnp.bfloat16), sa, dn,
                              preferred_element_type=f32)
    ha2 = jax.lax.dot_general(xc.astype(jnp.bfloat16), sa, dn,
                              preferred_element_type=f32)
    xbd = jnp.concatenate([xb, xd], axis=1)
    hb = jax.lax.dot_general(xbd.astype(jnp.bfloat16), sb, dn,
                             preferred_element_type=f32)

    h = jnp.concatenate(
        [ha1[:, 0:96, :],
         ha1[:, 96:_R, :] + hb[:, 0:32, :],
         ha2 + hb[:, 32:48, :]], axis=1)

    v = _vert16(h, _R + 16)[:, 0:_R, :]

    aa = jax.lax.broadcasted_iota(jnp.int32, (_R, _R), 0)
    ee = jax.lax.broadcasted_iota(jnp.int32, (_R, _R), 1)
    eye = (aa == ee)[None]
    return jnp.sum(jnp.where(eye, v, 0.0), axis=1)


def _diamond_kernel(xa0, xc0, xb0, xd0, xa1, xc1, xb1, xd1,
                    xa2, xc2, xb2, xd2, xa3, xc3, xb3, xd3, o_ref):
    cc = jax.lax.broadcasted_iota(jnp.int32, (_R, _R), 0)
    dd = jax.lax.broadcasted_iota(jnp.int32, (_R, _R), 1)
    ta = cc - dd
    sa = ((ta >= _DS + 1) & (ta < 2 * _DS + 1)).astype(jnp.bfloat16)
    tb = cc + _R - dd
    sb = ((tb >= _DS + 1) & (tb < 2 * _DS + 1)).astype(jnp.bfloat16)

    scale = 1.0 / (_DS * _DS)
    o_ref[:, 0:_R] = _chunk(xa0[...], xc0[...], xb0[...], xd0[...], sa, sb) * scale
    o_ref[:, _R:2 * _R] = _chunk(xa1[...], xc1[...], xb1[...], xd1[...], sa, sb) * scale
    o_ref[:, 2 * _R:3 * _R] = _chunk(xa2[...], xc2[...], xb2[...], xd2[...], sa, sb) * scale
    o_ref[:, 3 * _R:4 * _R] = _chunk(xa3[...], xc3[...], xb3[...], xd3[...], sa, sb) * scale


def _specs(k):
    return [
        pl.BlockSpec((_BB, _R, _R),
                     lambda i: (0, 4 * i + k, 4 * i + k)),
        pl.BlockSpec((_BB, 16, _R),
                     lambda i: (0, jnp.minimum(8 * (4 * i + k) + 8, _MAX16),
                                4 * i + k)),
        pl.BlockSpec((_BB, 32, _R),
                     lambda i: (0, 4 * (4 * i + k) + 3,
                                jnp.minimum(4 * i + k + 1, _MAXC))),
        pl.BlockSpec((_BB, 16, _R),
                     lambda i: (0, jnp.minimum(8 * (4 * i + k) + 8, _MAX16),
                                jnp.minimum(4 * i + k + 1, _MAXC))),
    ]


@jax.jit
def kernel(x):
    b = x.shape[0]
    grid = (_N_I // 4,)
    out = pl.pallas_call(
        _diamond_kernel,
        grid=grid,
        in_specs=_specs(0) + _specs(1) + _specs(2) + _specs(3),
        out_specs=pl.BlockSpec((_BB, 4 * _R), lambda i: (0, i)),
        out_shape=jax.ShapeDtypeStruct((b, _MAT), jnp.float32),
        compiler_params=pltpu.CompilerParams(
            dimension_semantics=("parallel",)),
    )(*([x] * 16))
    return out[:, :_ND]


# speedup vs baseline: 1.0692x; 1.0692x over previous
import jax
import jax.numpy as jnp
from jax.experimental import pallas as pl
from jax.experimental.pallas import tpu as pltpu

_DS = 16
_MAT = 2048
_ND = _MAT - 2 * _DS
_R = 128
_BB = 32

_N_I = _MAT // _R
_MAXC = _N_I - 1
_MAX16 = _MAT // 16 - 1


def _vert16(x, n):
    w = x[:, 0:n - 1, :] + x[:, 1:n, :]
    w = w[:, 0:n - 3, :] + w[:, 2:n - 1, :]
    w = w[:, 0:n - 7, :] + w[:, 4:n - 3, :]
    w = w[:, 0:n - 15, :] + w[:, 8:n - 7, :]
    return w


def _chunk(xa, xc, xb, xd, sa, sb):
    dn = (((2,), (0,)), ((), ()))
    f32 = jnp.float32
    ha1 = jax.lax.dot_general(xa.astype(jnp.bfloat16), sa, dn,
                              preferred_element_type=f32)
    ha2 = jax.lax.dot_general(xc.astype(jnp.bfloat16), sa, dn,
                              preferred_element_type=f32)
    xbd = jnp.concatenate([xb, xd], axis=1)
    hb = jax.lax.dot_general(xbd.astype(jnp.bfloat16), sb, dn,
                             preferred_element_type=f32)

    h = jnp.concatenate(
        [ha1[:, 0:96, :],
         ha1[:, 96:_R, :] + hb[:, 0:32, :],
         ha2 + hb[:, 32:48, :]], axis=1)

    v = _vert16(h, _R + 16)[:, 0:_R, :]

    aa = jax.lax.broadcasted_iota(jnp.int32, (_R, _R), 0)
    ee = jax.lax.broadcasted_iota(jnp.int32, (_R, _R), 1)
    eye = (aa == ee)[None]
    return jnp.sum(jnp.where(eye, v, 0.0), axis=1)


def _diamond_kernel(xa0, xc0, xb0, xa1, xc1, xb1, xd1, o_ref):
    cc = jax.lax.broadcasted_iota(jnp.int32, (_R, _R), 0)
    dd = jax.lax.broadcasted_iota(jnp.int32, (_R, _R), 1)
    ta = cc - dd
    sa = ((ta >= _DS + 1) & (ta < 2 * _DS + 1)).astype(jnp.bfloat16)
    tb = cc + _R - dd
    sb = ((tb >= _DS + 1) & (tb < 2 * _DS + 1)).astype(jnp.bfloat16)

    o0 = _chunk(xa0[...], xc0[...], xb0[...], xa1[:, 0:16, :], sa, sb)
    o1 = _chunk(xa1[...], xc1[...], xb1[...], xd1[...], sa, sb)
    scale = 1.0 / (_DS * _DS)
    o_ref[:, 0:_R] = o0 * scale
    o_ref[:, _R:2 * _R] = o1 * scale


def _specs(k, with_xd):
    specs = [
        pl.BlockSpec((_BB, _R, _R),
                     lambda i: (0, 2 * i + k, 2 * i + k)),
        pl.BlockSpec((_BB, 16, _R),
                     lambda i: (0, jnp.minimum(8 * (2 * i + k) + 8, _MAX16),
                                2 * i + k)),
        pl.BlockSpec((_BB, 32, _R),
                     lambda i: (0, 4 * (2 * i + k) + 3,
                                jnp.minimum(2 * i + k + 1, _MAXC))),
    ]
    if with_xd:
        specs.append(
            pl.BlockSpec((_BB, 16, _R),
                         lambda i: (0, jnp.minimum(8 * (2 * i + k) + 8, _MAX16),
                                    jnp.minimum(2 * i + k + 1, _MAXC))))
    return specs


@jax.jit
def kernel(x):
    b = x.shape[0]
    grid = (_N_I // 2,)
    out = pl.pallas_call(
        _diamond_kernel,
        grid=grid,
        in_specs=_specs(0, False) + _specs(1, True),
        out_specs=pl.BlockSpec((_BB, 2 * _R), lambda i: (0, i)),
        out_shape=jax.ShapeDtypeStruct((b, _MAT), jnp.float32),
        compiler_params=pltpu.CompilerParams(
            dimension_semantics=("parallel",)),
    )(x, x, x, x, x, x, x)
    return out[:, :_ND]
